# baseline (speedup 1.0000x reference)
import jax
import jax.numpy as jnp
from jax import lax
from jax.experimental import pallas as pl
from jax.experimental.pallas import tpu as pltpu
from jax.experimental.pallas import tpu_sc as plsc

B, L, F, R = 32, 256, 128, 32
FR = F * R
LANES = 16
NC, NS = 2, 16
NW = NC * NS

BT = 24
NSC = B - BT
K = NW // NSC
LW = L // K
CL = 8
NCHUNK = LW // CL
TBB = 4
CLT = L


def _sc_body(x_hbm, w1_hbm, w2_hbm, a1_hbm, a2_hbm,
             buf, wv1, wv2, acc1, acc2, sem0, sem1):
    wid = lax.axis_index("s") * NC + lax.axis_index("c")
    b = BT + wid // K
    lw0 = (wid % K) * LW

    pltpu.sync_copy(w1_hbm.at[pl.ds(b * 2, 2)], wv1)
    pltpu.sync_copy(w2_hbm.at[pl.ds(b * 2, 2)], wv2)

    zero = jnp.zeros((LANES,), jnp.float32)

    @plsc.parallel_loop(0, R * 8, step=1, unroll=4)
    def _zero_body(i):
        q = i >> 3
        c = (i & 7) * LANES
        acc1[q, pl.ds(c, LANES)] = zero
        acc2[q, pl.ds(c, LANES)] = zero

    sems = (sem0, sem1)

    def _chunk_copy(g, d):
        return pltpu.make_async_copy(
            x_hbm.at[b, pl.ds(lw0 + g * CL, CL)], buf.at[d], sems[d])

    _chunk_copy(0, 0).start()

    def _compute(d, w1s, w2s):
        @plsc.parallel_loop(0, R * 8, step=1, unroll=4)
        def _vbody(v):
            q = v >> 3
            c = (v & 7) * LANES
            xs = [buf[d, li, q, pl.ds(c, LANES)] for li in range(CL)]
            s1 = ((xs[0] * w1s[0] + xs[1] * w1s[1])
                  + (xs[2] * w1s[2] + xs[3] * w1s[3]))
            t1 = ((xs[4] * w1s[4] + xs[5] * w1s[5])
                  + (xs[6] * w1s[6] + xs[7] * w1s[7]))
            s2 = ((xs[0] * w2s[0] + xs[1] * w2s[1])
                  + (xs[2] * w2s[2] + xs[3] * w2s[3]))
            t2 = ((xs[4] * w2s[4] + xs[5] * w2s[5])
                  + (xs[6] * w2s[6] + xs[7] * w2s[7]))
            acc1[q, pl.ds(c, LANES)] = acc1[q, pl.ds(c, LANES)] + (s1 + t1)
            acc2[q, pl.ds(c, LANES)] = acc2[q, pl.ds(c, LANES)] + (s2 + t2)

    def _pair(gg, _):
        off = lw0 + gg * 2 * CL
        w1v = wv1[off >> 7, pl.ds(off & 127, LANES)]
        w2v = wv2[off >> 7, pl.ds(off & 127, LANES)]
        for d in range(2):
            g = gg * 2 + d
            w1s = [jnp.broadcast_to(w1v[d * CL + li], (LANES,))
                   for li in range(CL)]
            w2s = [jnp.broadcast_to(w2v[d * CL + li], (LANES,))
                   for li in range(CL)]

            @pl.when(g + 1 < NCHUNK)
            def _start_next():
                _chunk_copy(g + 1, 1 - d).start()

            _chunk_copy(g, d).wait()
            _compute(d, w1s, w2s)
        return 0

    lax.fori_loop(0, NCHUNK // 2, _pair, 0)

    out_row = wid * R
    pltpu.sync_copy(acc1, a1_hbm.at[pl.ds(out_row, R)])
    pltpu.sync_copy(acc2, a2_hbm.at[pl.ds(out_row, R)])


@jax.jit
def _sc_reduce(x4, w1, w2):
    mesh = plsc.VectorSubcoreMesh(core_axis_name="c", subcore_axis_name="s",
                                  num_cores=NC, num_subcores=NS)
    return pl.kernel(
        _sc_body,
        out_type=(jax.ShapeDtypeStruct((NW * R, 128), jnp.float32),
                  jax.ShapeDtypeStruct((NW * R, 128), jnp.float32)),
        mesh=mesh,
        scratch_types=(
            pltpu.VMEM((2, CL, R, 128), jnp.float32),
            pltpu.VMEM((2, 128), jnp.float32),
            pltpu.VMEM((2, 128), jnp.float32),
            pltpu.VMEM((R, 128), jnp.float32),
            pltpu.VMEM((R, 128), jnp.float32),
            pltpu.SemaphoreType.DMA,
            pltpu.SemaphoreType.DMA,
        ),
        name="cons_net_sc_reduce",
    )(x4, w1, w2)


def _tcr_body(w1s, w2s, x, cl, cr, rf, rr, out):
    b0 = pl.program_id(0) * TBB

    zero = jnp.zeros((R, F), jnp.float32)

    def _lbody(l, accs):
        new = []
        for j in range(TBB):
            xl = x[j, l]
            new.append(accs[2 * j] + w1s[b0 + j, l] * xl)
            new.append(accs[2 * j + 1] + w2s[b0 + j, l] * xl)
        return tuple(new)

    accs = lax.fori_loop(0, CLT, _lbody, (zero,) * (2 * TBB), unroll=4)

    clv = cl[...]
    crv = cr[...]
    rrv = rr[...]
    for j in range(TBB):
        o = jnp.dot(clv, accs[2 * j], preferred_element_type=jnp.float32)
        o = o + jnp.dot(crv, accs[2 * j + 1],
                        preferred_element_type=jnp.float32)
        out[j] = o + rrv * rf[pl.ds(b0 + j, 1), :]


@jax.jit
def _tc_reduce(x4, w1, w2, cl, cr, rf, rr):
    smem_full = pl.BlockSpec((B, L), lambda i: (0, 0),
                             memory_space=pltpu.SMEM)
    full = lambda shape: pl.BlockSpec(shape, lambda i: (0,) * len(shape))
    return pl.pallas_call(
        _tcr_body,
        grid=(BT // TBB,),
        in_specs=[smem_full, smem_full,
                  pl.BlockSpec((TBB, CLT, R, F), lambda i: (i, 0, 0, 0)),
                  full((R, R)), full((R, R)), full((B, F)), full((R, 1))],
        out_specs=pl.BlockSpec((TBB, R, F), lambda i: (i, 0, 0)),
        out_shape=jax.ShapeDtypeStruct((BT, R, F), jnp.float32),
        name="cons_net_tc_reduce",
    )(w1, w2, x4, cl, cr, rf, rr)


def _epi_body(a1p, a2p, cl, cr, rf, rr, w1, w2, out, m1, m2):
    clv = cl[...]
    crv = cr[...]
    rrv = rr[...]

    def _bs(b, _):
        base = b * K * R
        p1 = a1p[pl.ds(base, R), :]
        p2 = a2p[pl.ds(base, R), :]
        for j in range(1, K):
            p1 = p1 + a1p[pl.ds(base + j * R, R), :]
            p2 = p2 + a2p[pl.ds(base + j * R, R), :]
        o = jnp.dot(clv, p1, preferred_element_type=jnp.float32)
        o = o + jnp.dot(crv, p2, preferred_element_type=jnp.float32)
        out[pl.ds(b * R, R), :] = o + rrv * rf[pl.ds(BT + b, 1), :]
        return 0

    lax.fori_loop(0, NSC, _bs, 0)
    m1[...] = jnp.max(w1[...], axis=1, keepdims=True)
    m2[...] = jnp.max(w2[...], axis=1, keepdims=True)


@jax.jit
def _sc_epilogue(a1p, a2p, cl, cr, rf, rr, w1, w2):
    return pl.pallas_call(
        _epi_body,
        out_shape=(jax.ShapeDtypeStruct((NSC * R, F), jnp.float32),
                   jax.ShapeDtypeStruct((B, 1), jnp.float32),
                   jax.ShapeDtypeStruct((B, 1), jnp.float32)),
        name="cons_net_sc_epilogue",
    )(a1p, a2p, cl, cr, rf, rr, w1, w2)


def kernel(x, arg1_weight, arg2_weight, root_filler, cons_l, cons_r, root_role):
    x4 = x.transpose(0, 1, 3, 2)
    w1_2d = arg1_weight.reshape(B * L // 128, 128)
    w2_2d = arg2_weight.reshape(B * L // 128, 128)
    rr = root_role.reshape(R, 1)

    a1p, a2p = _sc_reduce(x4, w1_2d, w2_2d)
    out_tc = _tc_reduce(x4, arg1_weight, arg2_weight, cons_l, cons_r,
                        root_filler, rr)
    out_sc, m1, m2 = _sc_epilogue(a1p, a2p, cons_l, cons_r, root_filler, rr,
                                  arg1_weight, arg2_weight)
    out = jnp.concatenate([out_tc.reshape(BT * R, F), out_sc], axis=0)
    return (out.reshape(B, R, F).transpose(0, 2, 1),
            m1.reshape(B), m2.reshape(B))


# speedup vs baseline: 1.0635x; 1.0635x over previous
import jax
import jax.numpy as jnp
from jax import lax
from jax.experimental import pallas as pl
from jax.experimental.pallas import tpu as pltpu
from jax.experimental.pallas import tpu_sc as plsc

B, L, F, R = 32, 256, 128, 32
FR = F * R
LANES = 16
NC, NS = 2, 16
NW = NC * NS

BT = 28
NSC = B - BT
K = NW // NSC
LW = L // K
CL = 8
NCHUNK = LW // CL
TBB = 4
CLT = L


def _sc_body(x_hbm, w1_hbm, w2_hbm, a1_hbm, a2_hbm,
             buf, wv1, wv2, acc1, acc2, sem0, sem1):
    wid = lax.axis_index("s") * NC + lax.axis_index("c")
    b = BT + wid // K
    lw0 = (wid % K) * LW

    pltpu.sync_copy(w1_hbm.at[b >> 3, :, b & 7], wv1)
    pltpu.sync_copy(w2_hbm.at[b >> 3, :, b & 7], wv2)

    zero = jnp.zeros((LANES,), jnp.float32)

    @plsc.parallel_loop(0, R * 8, step=1, unroll=4)
    def _zero_body(i):
        q = i >> 3
        c = (i & 7) * LANES
        acc1[q, pl.ds(c, LANES)] = zero
        acc2[q, pl.ds(c, LANES)] = zero

    sems = (sem0, sem1)

    def _chunk_copy(g, d):
        return pltpu.make_async_copy(
            x_hbm.at[b, pl.ds(lw0 + g * CL, CL)], buf.at[d], sems[d])

    _chunk_copy(0, 0).start()

    def _compute(d, w1s, w2s):
        @plsc.parallel_loop(0, R * 8, step=1, unroll=4)
        def _vbody(v):
            q = v >> 3
            c = (v & 7) * LANES
            xs = [buf[d, li, q, pl.ds(c, LANES)] for li in range(CL)]
            s1 = ((xs[0] * w1s[0] + xs[1] * w1s[1])
                  + (xs[2] * w1s[2] + xs[3] * w1s[3]))
            t1 = ((xs[4] * w1s[4] + xs[5] * w1s[5])
                  + (xs[6] * w1s[6] + xs[7] * w1s[7]))
            s2 = ((xs[0] * w2s[0] + xs[1] * w2s[1])
                  + (xs[2] * w2s[2] + xs[3] * w2s[3]))
            t2 = ((xs[4] * w2s[4] + xs[5] * w2s[5])
                  + (xs[6] * w2s[6] + xs[7] * w2s[7]))
            acc1[q, pl.ds(c, LANES)] = acc1[q, pl.ds(c, LANES)] + (s1 + t1)
            acc2[q, pl.ds(c, LANES)] = acc2[q, pl.ds(c, LANES)] + (s2 + t2)

    def _pair(gg, _):
        off = lw0 + gg * 2 * CL
        w1v = wv1[off >> 7, pl.ds(off & 127, LANES)]
        w2v = wv2[off >> 7, pl.ds(off & 127, LANES)]
        for d in range(2):
            g = gg * 2 + d
            w1s = [jnp.broadcast_to(w1v[d * CL + li], (LANES,))
                   for li in range(CL)]
            w2s = [jnp.broadcast_to(w2v[d * CL + li], (LANES,))
                   for li in range(CL)]

            @pl.when(g + 1 < NCHUNK)
            def _start_next():
                _chunk_copy(g + 1, 1 - d).start()

            _chunk_copy(g, d).wait()
            _compute(d, w1s, w2s)
        return 0

    lax.fori_loop(0, NCHUNK // 2, _pair, 0)

    out_row = wid * R
    pltpu.sync_copy(acc1, a1_hbm.at[pl.ds(out_row, R)])
    pltpu.sync_copy(acc2, a2_hbm.at[pl.ds(out_row, R)])


@jax.jit
def _sc_reduce(x4, w1, w2):
    mesh = plsc.VectorSubcoreMesh(core_axis_name="c", subcore_axis_name="s",
                                  num_cores=NC, num_subcores=NS)
    return pl.kernel(
        _sc_body,
        out_type=(jax.ShapeDtypeStruct((NW * R, 128), jnp.float32),
                  jax.ShapeDtypeStruct((NW * R, 128), jnp.float32)),
        mesh=mesh,
        scratch_types=(
            pltpu.VMEM((2, CL, R, 128), jnp.float32),
            pltpu.VMEM((2, 128), jnp.float32),
            pltpu.VMEM((2, 128), jnp.float32),
            pltpu.VMEM((R, 128), jnp.float32),
            pltpu.VMEM((R, 128), jnp.float32),
            pltpu.SemaphoreType.DMA,
            pltpu.SemaphoreType.DMA,
        ),
        name="cons_net_sc_reduce",
    )(x4, w1, w2)


def _tcr_body(w1s, w2s, x, cl, cr, rf, rr, out):
    b0 = pl.program_id(0) * TBB

    zero = jnp.zeros((R, F), jnp.float32)

    def _lbody(l, accs):
        new = []
        for j in range(TBB):
            xl = x[j, l]
            new.append(accs[2 * j] + w1s[b0 + j, l] * xl)
            new.append(accs[2 * j + 1] + w2s[b0 + j, l] * xl)
        return tuple(new)

    accs = lax.fori_loop(0, CLT, _lbody, (zero,) * (2 * TBB), unroll=4)

    clv = cl[...]
    crv = cr[...]
    rrv = rr[...]
    for j in range(TBB):
        o = jnp.dot(clv, accs[2 * j], preferred_element_type=jnp.float32)
        o = o + jnp.dot(crv, accs[2 * j + 1],
                        preferred_element_type=jnp.float32)
        out[j] = o + rrv * rf[pl.ds(b0 + j, 1), :]


@jax.jit
def _tc_reduce(x4, w1, w2, cl, cr, rf, rr):
    smem_full = pl.BlockSpec((B, L), lambda i: (0, 0),
                             memory_space=pltpu.SMEM)
    full = lambda shape: pl.BlockSpec(shape, lambda i: (0,) * len(shape))
    return pl.pallas_call(
        _tcr_body,
        grid=(BT // TBB,),
        in_specs=[smem_full, smem_full,
                  pl.BlockSpec((TBB, CLT, R, F), lambda i: (i, 0, 0, 0)),
                  full((R, R)), full((R, R)), full((B, F)), full((R, 1))],
        out_specs=pl.BlockSpec((TBB, R, F), lambda i: (i, 0, 0)),
        out_shape=jax.ShapeDtypeStruct((BT, R, F), jnp.float32),
        name="cons_net_tc_reduce",
    )(w1, w2, x4, cl, cr, rf, rr)


def _epi_body(a1p, a2p, cl, cr, rf, rr, w1, w2, out, m1, m2):
    clv = cl[...]
    crv = cr[...]
    rrv = rr[...]

    def _bs(b, _):
        base = b * K * R
        p1 = a1p[pl.ds(base, R), :]
        p2 = a2p[pl.ds(base, R), :]
        for j in range(1, K):
            p1 = p1 + a1p[pl.ds(base + j * R, R), :]
            p2 = p2 + a2p[pl.ds(base + j * R, R), :]
        o = jnp.dot(clv, p1, preferred_element_type=jnp.float32)
        o = o + jnp.dot(crv, p2, preferred_element_type=jnp.float32)
        out[pl.ds(b * R, R), :] = o + rrv * rf[pl.ds(BT + b, 1), :]
        return 0

    lax.fori_loop(0, NSC, _bs, 0)
    m1[...] = jnp.max(w1[...], axis=1, keepdims=True)
    m2[...] = jnp.max(w2[...], axis=1, keepdims=True)


@jax.jit
def _sc_epilogue(a1p, a2p, cl, cr, rf, rr, w1, w2):
    return pl.pallas_call(
        _epi_body,
        out_shape=(jax.ShapeDtypeStruct((NSC * R, F), jnp.float32),
                   jax.ShapeDtypeStruct((B, 1), jnp.float32),
                   jax.ShapeDtypeStruct((B, 1), jnp.float32)),
        name="cons_net_sc_epilogue",
    )(a1p, a2p, cl, cr, rf, rr, w1, w2)


def kernel(x, arg1_weight, arg2_weight, root_filler, cons_l, cons_r, root_role):
    x4 = x.transpose(0, 1, 3, 2)
    w1_4d = arg1_weight.reshape(4, 8, 2, 128).transpose(0, 2, 1, 3)
    w2_4d = arg2_weight.reshape(4, 8, 2, 128).transpose(0, 2, 1, 3)
    rr = root_role.reshape(R, 1)

    a1p, a2p = _sc_reduce(x4, w1_4d, w2_4d)
    out_tc = _tc_reduce(x4, arg1_weight, arg2_weight, cons_l, cons_r,
                        root_filler, rr)
    out_sc, m1, m2 = _sc_epilogue(a1p, a2p, cons_l, cons_r, root_filler, rr,
                                  arg1_weight, arg2_weight)
    out = jnp.concatenate([out_tc.reshape(BT * R, F), out_sc], axis=0)
    return (out.reshape(B, R, F).transpose(0, 2, 1),
            m1.reshape(B), m2.reshape(B))


# speedup vs baseline: 1.1021x; 1.0362x over previous
import jax
import jax.numpy as jnp
from jax import lax
from jax.experimental import pallas as pl
from jax.experimental.pallas import tpu as pltpu
from jax.experimental.pallas import tpu_sc as plsc

B, L, F, R = 32, 256, 128, 32
FR = F * R
LANES = 16
NC, NS = 2, 16
NW = NC * NS

BT = 28
NSC = B - BT
K = NW // NSC
LW = L // K
CL = 8
NCHUNK = LW // CL
TBB = 4
CLT = L


def _sc_body(x_hbm, w1_hbm, w2_hbm, a1_hbm, a2_hbm,
             buf, wv1, wv2, acc1, acc2, sem0, sem1):
    wid = lax.axis_index("s") * NC + lax.axis_index("c")
    b = BT + wid // K
    lw0 = (wid % K) * LW

    pltpu.sync_copy(w1_hbm.at[b >> 3, :, b & 7], wv1)
    pltpu.sync_copy(w2_hbm.at[b >> 3, :, b & 7], wv2)

    zero = jnp.zeros((LANES,), jnp.float32)

    @plsc.parallel_loop(0, R * 8, step=1, unroll=4)
    def _zero_body(i):
        q = i >> 3
        c = (i & 7) * LANES
        acc1[q, pl.ds(c, LANES)] = zero
        acc2[q, pl.ds(c, LANES)] = zero

    sems = (sem0, sem1)

    def _chunk_copy(g, d):
        return pltpu.make_async_copy(
            x_hbm.at[b, pl.ds(lw0 + g * CL, CL)], buf.at[d], sems[d])

    _chunk_copy(0, 0).start()

    def _compute(d, w1s, w2s):
        @plsc.parallel_loop(0, R * 8, step=1, unroll=4)
        def _vbody(v):
            q = v >> 3
            c = (v & 7) * LANES
            xs = [buf[d, li, q, pl.ds(c, LANES)] for li in range(CL)]
            s1 = ((xs[0] * w1s[0] + xs[1] * w1s[1])
                  + (xs[2] * w1s[2] + xs[3] * w1s[3]))
            t1 = ((xs[4] * w1s[4] + xs[5] * w1s[5])
                  + (xs[6] * w1s[6] + xs[7] * w1s[7]))
            s2 = ((xs[0] * w2s[0] + xs[1] * w2s[1])
                  + (xs[2] * w2s[2] + xs[3] * w2s[3]))
            t2 = ((xs[4] * w2s[4] + xs[5] * w2s[5])
                  + (xs[6] * w2s[6] + xs[7] * w2s[7]))
            acc1[q, pl.ds(c, LANES)] = acc1[q, pl.ds(c, LANES)] + (s1 + t1)
            acc2[q, pl.ds(c, LANES)] = acc2[q, pl.ds(c, LANES)] + (s2 + t2)

    def _pair(gg, _):
        off = lw0 + gg * 2 * CL
        w1v = wv1[off >> 7, pl.ds(off & 127, LANES)]
        w2v = wv2[off >> 7, pl.ds(off & 127, LANES)]
        for d in range(2):
            g = gg * 2 + d
            w1s = [jnp.broadcast_to(w1v[d * CL + li], (LANES,))
                   for li in range(CL)]
            w2s = [jnp.broadcast_to(w2v[d * CL + li], (LANES,))
                   for li in range(CL)]

            @pl.when(g + 1 < NCHUNK)
            def _start_next():
                _chunk_copy(g + 1, 1 - d).start()

            _chunk_copy(g, d).wait()
            _compute(d, w1s, w2s)
        return 0

    lax.fori_loop(0, NCHUNK // 2, _pair, 0)

    out_row = wid * R
    pltpu.sync_copy(acc1, a1_hbm.at[pl.ds(out_row, R)])
    pltpu.sync_copy(acc2, a2_hbm.at[pl.ds(out_row, R)])


@jax.jit
def _sc_reduce(x4, w1, w2):
    mesh = plsc.VectorSubcoreMesh(core_axis_name="c", subcore_axis_name="s",
                                  num_cores=NC, num_subcores=NS)
    return pl.kernel(
        _sc_body,
        out_type=(jax.ShapeDtypeStruct((NW * R, 128), jnp.float32),
                  jax.ShapeDtypeStruct((NW * R, 128), jnp.float32)),
        mesh=mesh,
        scratch_types=(
            pltpu.VMEM((2, CL, R, 128), jnp.float32),
            pltpu.VMEM((2, 128), jnp.float32),
            pltpu.VMEM((2, 128), jnp.float32),
            pltpu.VMEM((R, 128), jnp.float32),
            pltpu.VMEM((R, 128), jnp.float32),
            pltpu.SemaphoreType.DMA,
            pltpu.SemaphoreType.DMA,
        ),
        name="cons_net_sc_reduce",
    )(x4, w1, w2)


def _tcr_body(w1s, w2s, x, cl, cr, rf, rr, out):
    b0 = pl.program_id(0) * TBB

    zero = jnp.zeros((R, F), jnp.float32)

    def _lbody(l, accs):
        new = []
        for j in range(TBB):
            xl = x[j, l]
            new.append(accs[2 * j] + w1s[b0 + j, l] * xl)
            new.append(accs[2 * j + 1] + w2s[b0 + j, l] * xl)
        return tuple(new)

    accs = lax.fori_loop(0, CLT, _lbody, (zero,) * (2 * TBB), unroll=4)

    clv = cl[...]
    crv = cr[...]
    rrv = rr[...]
    for j in range(TBB):
        o = jnp.dot(clv, accs[2 * j], preferred_element_type=jnp.float32)
        o = o + jnp.dot(crv, accs[2 * j + 1],
                        preferred_element_type=jnp.float32)
        out[j] = o + rrv * rf[pl.ds(b0 + j, 1), :]


@jax.jit
def _tc_reduce(x4, w1, w2, cl, cr, rf, rr):
    smem_full = pl.BlockSpec((B, L), lambda i: (0, 0),
                             memory_space=pltpu.SMEM)
    full = lambda shape: pl.BlockSpec(shape, lambda i: (0,) * len(shape))
    return pl.pallas_call(
        _tcr_body,
        grid=(BT // TBB,),
        in_specs=[smem_full, smem_full,
                  pl.BlockSpec((TBB, CLT, R, F), lambda i: (i, 0, 0, 0)),
                  full((R, R)), full((R, R)), full((B, F)), full((R, 1))],
        out_specs=pl.BlockSpec((TBB, R, F), lambda i: (i, 0, 0)),
        out_shape=jax.ShapeDtypeStruct((BT, R, F), jnp.float32),
        name="cons_net_tc_reduce",
    )(w1, w2, x4, cl, cr, rf, rr)


def _epi_body(a1p, a2p, cl, cr, rf, rr, w1, w2, out, m1, m2):
    clv = cl[...]
    crv = cr[...]
    rrv = rr[...]

    def _bs(b, _):
        base = b * K * R
        p1 = a1p[pl.ds(base, R), :]
        p2 = a2p[pl.ds(base, R), :]
        for j in range(1, K):
            p1 = p1 + a1p[pl.ds(base + j * R, R), :]
            p2 = p2 + a2p[pl.ds(base + j * R, R), :]
        o = jnp.dot(clv, p1, preferred_element_type=jnp.float32)
        o = o + jnp.dot(crv, p2, preferred_element_type=jnp.float32)
        out[pl.ds(b * R, R), :] = o + rrv * rf[pl.ds(BT + b, 1), :]
        return 0

    lax.fori_loop(0, NSC, _bs, 0)
    m1[...] = jnp.max(w1[...], axis=1)
    m2[...] = jnp.max(w2[...], axis=1)


@jax.jit
def _sc_epilogue(a1p, a2p, cl, cr, rf, rr, w1, w2):
    return pl.pallas_call(
        _epi_body,
        out_shape=(jax.ShapeDtypeStruct((NSC * R, F), jnp.float32),
                   jax.ShapeDtypeStruct((B,), jnp.float32),
                   jax.ShapeDtypeStruct((B,), jnp.float32)),
        name="cons_net_sc_epilogue",
    )(a1p, a2p, cl, cr, rf, rr, w1, w2)


def kernel(x, arg1_weight, arg2_weight, root_filler, cons_l, cons_r, root_role):
    x4 = x.transpose(0, 1, 3, 2)
    w1_4d = arg1_weight.reshape(4, 8, 2, 128).transpose(0, 2, 1, 3)
    w2_4d = arg2_weight.reshape(4, 8, 2, 128).transpose(0, 2, 1, 3)
    rr = root_role.reshape(R, 1)

    a1p, a2p = _sc_reduce(x4, w1_4d, w2_4d)
    out_tc = _tc_reduce(x4, arg1_weight, arg2_weight, cons_l, cons_r,
                        root_filler, rr)
    out_sc, m1, m2 = _sc_epilogue(a1p, a2p, cons_l, cons_r, root_filler, rr,
                                  arg1_weight, arg2_weight)
    out = jnp.concatenate([out_tc.reshape(BT * R, F), out_sc], axis=0)
    return out.reshape(B, R, F).transpose(0, 2, 1), m1, m2


# speedup vs baseline: 1.1225x; 1.0186x over previous
import jax
import jax.numpy as jnp
from jax import lax
from jax.experimental import pallas as pl
from jax.experimental.pallas import tpu as pltpu
from jax.experimental.pallas import tpu_sc as plsc

B, L, F, R = 32, 256, 128, 32
FR = F * R
LANES = 16
NC, NS = 2, 16
NW = NC * NS

BT = 28
NSC = B - BT
K = NW // NSC
LW = L // K
CL = 8
NCHUNK = LW // CL
TBB = 4
CLT = L


def _sc_body(x_hbm, w1_hbm, w2_hbm, a1_hbm, a2_hbm,
             buf, wv1, wv2, acc1, acc2, sem0, sem1):
    wid = lax.axis_index("s") * NC + lax.axis_index("c")
    b = BT + wid // K
    lw0 = (wid % K) * LW

    pltpu.sync_copy(w1_hbm.at[b >> 3, :, b & 7], wv1)
    pltpu.sync_copy(w2_hbm.at[b >> 3, :, b & 7], wv2)

    zero = jnp.zeros((LANES,), jnp.float32)

    @plsc.parallel_loop(0, R * 8, step=1, unroll=4)
    def _zero_body(i):
        q = i >> 3
        c = (i & 7) * LANES
        acc1[q, pl.ds(c, LANES)] = zero
        acc2[q, pl.ds(c, LANES)] = zero

    sems = (sem0, sem1)

    def _chunk_copy(g, d):
        return pltpu.make_async_copy(
            x_hbm.at[b, pl.ds(lw0 + g * CL, CL)], buf.at[d], sems[d])

    _chunk_copy(0, 0).start()

    def _compute(d, w1s, w2s):
        @plsc.parallel_loop(0, R * 8, step=1, unroll=4)
        def _vbody(v):
            q = v >> 3
            c = (v & 7) * LANES
            xs = [buf[d, li, q, pl.ds(c, LANES)] for li in range(CL)]
            s1 = ((xs[0] * w1s[0] + xs[1] * w1s[1])
                  + (xs[2] * w1s[2] + xs[3] * w1s[3]))
            t1 = ((xs[4] * w1s[4] + xs[5] * w1s[5])
                  + (xs[6] * w1s[6] + xs[7] * w1s[7]))
            s2 = ((xs[0] * w2s[0] + xs[1] * w2s[1])
                  + (xs[2] * w2s[2] + xs[3] * w2s[3]))
            t2 = ((xs[4] * w2s[4] + xs[5] * w2s[5])
                  + (xs[6] * w2s[6] + xs[7] * w2s[7]))
            acc1[q, pl.ds(c, LANES)] = acc1[q, pl.ds(c, LANES)] + (s1 + t1)
            acc2[q, pl.ds(c, LANES)] = acc2[q, pl.ds(c, LANES)] + (s2 + t2)

    def _pair(gg, _):
        off = lw0 + gg * 2 * CL
        w1v = wv1[off >> 7, pl.ds(off & 127, LANES)]
        w2v = wv2[off >> 7, pl.ds(off & 127, LANES)]
        for d in range(2):
            g = gg * 2 + d
            w1s = [jnp.broadcast_to(w1v[d * CL + li], (LANES,))
                   for li in range(CL)]
            w2s = [jnp.broadcast_to(w2v[d * CL + li], (LANES,))
                   for li in range(CL)]

            @pl.when(g + 1 < NCHUNK)
            def _start_next():
                _chunk_copy(g + 1, 1 - d).start()

            _chunk_copy(g, d).wait()
            _compute(d, w1s, w2s)
        return 0

    lax.fori_loop(0, NCHUNK // 2, _pair, 0)

    out_row = wid * R
    pltpu.sync_copy(acc1, a1_hbm.at[pl.ds(out_row, R)])
    pltpu.sync_copy(acc2, a2_hbm.at[pl.ds(out_row, R)])


@jax.jit
def _sc_reduce(x4, w1, w2):
    mesh = plsc.VectorSubcoreMesh(core_axis_name="c", subcore_axis_name="s",
                                  num_cores=NC, num_subcores=NS)
    return pl.kernel(
        _sc_body,
        out_type=(jax.ShapeDtypeStruct((NW * R, 128), jnp.float32),
                  jax.ShapeDtypeStruct((NW * R, 128), jnp.float32)),
        mesh=mesh,
        scratch_types=(
            pltpu.VMEM((2, CL, R, 128), jnp.float32),
            pltpu.VMEM((2, 128), jnp.float32),
            pltpu.VMEM((2, 128), jnp.float32),
            pltpu.VMEM((R, 128), jnp.float32),
            pltpu.VMEM((R, 128), jnp.float32),
            pltpu.SemaphoreType.DMA,
            pltpu.SemaphoreType.DMA,
        ),
        name="cons_net_sc_reduce",
    )(x4, w1, w2)


def _tcr_body(w1s, w2s, x, cl, cr, rf, rr, out):
    b0 = pl.program_id(0) * TBB

    zero = jnp.zeros((R, F), jnp.float32)

    def _lbody(l, accs):
        new = []
        for j in range(TBB):
            xl = x[j, l]
            new.append(accs[2 * j] + w1s[b0 + j, l] * xl)
            new.append(accs[2 * j + 1] + w2s[b0 + j, l] * xl)
        return tuple(new)

    accs = lax.fori_loop(0, CLT, _lbody, (zero,) * (2 * TBB), unroll=4)

    clv = cl[...]
    crv = cr[...]
    rrv = rr[...]
    for j in range(TBB):
        o = jnp.dot(clv, accs[2 * j], preferred_element_type=jnp.float32)
        o = o + jnp.dot(crv, accs[2 * j + 1],
                        preferred_element_type=jnp.float32)
        out[j] = o + rrv * rf[pl.ds(b0 + j, 1), :]


@jax.jit
def _tc_reduce(x4, w1, w2, cl, cr, rf, rr):
    smem_full = pl.BlockSpec((B, L), lambda i: (0, 0),
                             memory_space=pltpu.SMEM)
    full = lambda shape: pl.BlockSpec(shape, lambda i: (0,) * len(shape))
    return pl.pallas_call(
        _tcr_body,
        grid=(BT // TBB,),
        in_specs=[smem_full, smem_full,
                  pl.BlockSpec((TBB, CLT, R, F), lambda i: (i, 0, 0, 0)),
                  full((R, R)), full((R, R)), full((B, F)), full((R, 1))],
        out_specs=pl.BlockSpec((TBB, R, F), lambda i: (i, 0, 0)),
        out_shape=jax.ShapeDtypeStruct((BT, R, F), jnp.float32),
        name="cons_net_tc_reduce",
    )(w1, w2, x4, cl, cr, rf, rr)


def _epi_body(otc, a1p, a2p, cl, cr, rf, rr, w1, w2, out, m1, m2):
    out[pl.ds(0, BT * R), :] = otc[...]
    clv = cl[...]
    crv = cr[...]
    rrv = rr[...]

    def _bs(b, _):
        base = b * K * R
        p1 = a1p[pl.ds(base, R), :]
        p2 = a2p[pl.ds(base, R), :]
        for j in range(1, K):
            p1 = p1 + a1p[pl.ds(base + j * R, R), :]
            p2 = p2 + a2p[pl.ds(base + j * R, R), :]
        o = jnp.dot(clv, p1, preferred_element_type=jnp.float32)
        o = o + jnp.dot(crv, p2, preferred_element_type=jnp.float32)
        out[pl.ds(BT * R + b * R, R), :] = o + rrv * rf[pl.ds(BT + b, 1), :]
        return 0

    lax.fori_loop(0, NSC, _bs, 0)
    m1[...] = jnp.max(w1[...], axis=1)
    m2[...] = jnp.max(w2[...], axis=1)


@jax.jit
def _sc_epilogue(otc, a1p, a2p, cl, cr, rf, rr, w1, w2):
    return pl.pallas_call(
        _epi_body,
        out_shape=(jax.ShapeDtypeStruct((B * R, F), jnp.float32),
                   jax.ShapeDtypeStruct((B,), jnp.float32),
                   jax.ShapeDtypeStruct((B,), jnp.float32)),
        name="cons_net_sc_epilogue",
    )(otc, a1p, a2p, cl, cr, rf, rr, w1, w2)


def kernel(x, arg1_weight, arg2_weight, root_filler, cons_l, cons_r, root_role):
    x4 = x.transpose(0, 1, 3, 2)
    w1_4d = arg1_weight.reshape(4, 8, 2, 128).transpose(0, 2, 1, 3)
    w2_4d = arg2_weight.reshape(4, 8, 2, 128).transpose(0, 2, 1, 3)
    rr = root_role.reshape(R, 1)

    a1p, a2p = _sc_reduce(x4, w1_4d, w2_4d)
    out_tc = _tc_reduce(x4, arg1_weight, arg2_weight, cons_l, cons_r,
                        root_filler, rr)
    out, m1, m2 = _sc_epilogue(out_tc.reshape(BT * R, F), a1p, a2p,
                               cons_l, cons_r, root_filler, rr,
                               arg1_weight, arg2_weight)
    return out.reshape(B, R, F).transpose(0, 2, 1), m1, m2
